# baseline (speedup 1.0000x reference)
import functools

import jax
import jax.numpy as jnp
from jax import lax
from jax.experimental import pallas as pl
from jax.experimental.pallas import tpu as pltpu
from jax.experimental.pallas import tpu_sc as plsc

N_USERS = 25000
N_ITEMS = 25000
NN = N_USERS + 1 + N_ITEMS + 1
NP = 50176
D = 64
E = 800000
E_PAD = 819200
THRESHOLD = 0.05

NC, NS = 2, 16
NW = NC * NS
IB = 128

G_PER = E_PAD // NW
G_CH = 512
G_T = G_PER // G_CH
S_PER = E_PAD // NS
S_CH = 256
S_K = S_CH // IB
S_T = S_PER // S_CH
RP = NP // NS
D_CH = 224

_mesh = plsc.VectorSubcoreMesh(core_axis_name="c", subcore_axis_name="s")
_sc_params = pltpu.CompilerParams(use_tc_tiling_on_sc=False)


@functools.partial(
    pl.kernel, mesh=_mesh, compiler_params=_sc_params,
    out_type=jax.ShapeDtypeStruct((E_PAD, D), jnp.float32),
    scratch_types=[
        pltpu.VMEM((2, G_CH), jnp.int32),
        pltpu.VMEM((2, G_CH, D), jnp.float32),
        pltpu.SemaphoreType.DMA((2,)),
    ],
)
def _sc_gather(table_hbm, idx1_hbm, out_hbm, idx_v, rows_v, sems):
    wid = lax.axis_index("s") * NC + lax.axis_index("c")
    ebase = wid * G_PER

    def fire(g, b):
        pltpu.sync_copy(idx1_hbm.at[pl.ds(ebase + g * G_CH, G_CH)], idx_v.at[b])
        pltpu.async_copy(table_hbm.at[idx_v.at[b]], rows_v.at[b], sems.at[b])

    def drain_wb(g, b):
        off = ebase + g * G_CH
        pltpu.make_async_copy(out_hbm.at[pl.ds(off, G_CH)], rows_v.at[b],
                              sems.at[b]).wait()
        pltpu.sync_copy(rows_v.at[b], out_hbm.at[pl.ds(off, G_CH)])

    fire(0, 0)

    def body(t, carry):
        g = 2 * t
        fire(g + 1, 1)
        drain_wb(g, 0)
        fire(lax.rem(g + 2, G_T), 0)
        drain_wb(g + 1, 1)
        return carry

    lax.fori_loop(0, G_T // 2, body, 0)
    pltpu.make_async_copy(out_hbm.at[pl.ds(ebase, G_CH)], rows_v.at[0],
                          sems.at[0]).wait()


@functools.partial(
    pl.kernel, mesh=_mesh, compiler_params=_sc_params,
    out_type=jax.ShapeDtypeStruct((NC, NP, 32), jnp.float32),
    scratch_types=[
        pltpu.VMEM((2, S_K, IB), jnp.int32),
        pltpu.VMEM((2, S_CH, 32), jnp.float32),
        pltpu.VMEM_SHARED((NP, 32), jnp.float32),
        pltpu.SemaphoreType.DMA((2,)),
        pltpu.SemaphoreType.DMA((2,)),
    ],
)
def _sc_scatter(msg_hbm, dst2_hbm, zero_hbm, out_hbm, idx_v, msg_v,
                shared, lsem, ssem):
    c = lax.axis_index("c")
    s = lax.axis_index("s")
    pltpu.sync_copy(zero_hbm.at[pl.ds(s * RP, RP)], shared.at[pl.ds(s * RP, RP)])
    plsc.subcore_barrier()

    rbase = s * (S_PER // IB)
    ebase = s * S_PER

    def fire(g, b):
        pltpu.sync_copy(dst2_hbm.at[pl.ds(rbase + g * S_K, S_K)], idx_v.at[b])
        pltpu.async_copy(msg_hbm.at[c, pl.ds(ebase + g * S_CH, S_CH)],
                         msg_v.at[b], lsem.at[b])

    def process(b):
        pltpu.make_async_copy(msg_hbm.at[c, pl.ds(ebase, S_CH)], msg_v.at[b],
                              lsem.at[b]).wait()
        for j in range(S_K):
            pltpu.async_copy(msg_v.at[b].at[pl.ds(j * IB, IB)],
                             shared.at[idx_v.at[b].at[j]], ssem.at[b],
                             add=True)
        for j in range(S_K):
            pltpu.make_async_copy(msg_v.at[b].at[pl.ds(j * IB, IB)],
                                  shared.at[idx_v.at[b].at[j]],
                                  ssem.at[b]).wait()

    fire(0, 0)

    def body(t, carry):
        g = 2 * t
        fire(g + 1, 1)
        process(0)
        fire(lax.rem(g + 2, S_T), 0)
        process(1)
        return carry

    lax.fori_loop(0, S_T // 2, body, 0)
    pltpu.make_async_copy(msg_hbm.at[c, pl.ds(ebase, S_CH)], msg_v.at[0],
                          lsem.at[0]).wait()
    plsc.subcore_barrier()

    def dump(jj, carry):
        r = s * RP + jj * D_CH
        pltpu.sync_copy(shared.at[pl.ds(r, D_CH)], msg_v.at[0, pl.ds(0, D_CH)])
        pltpu.sync_copy(msg_v.at[0, pl.ds(0, D_CH)],
                        out_hbm.at[c, pl.ds(r, D_CH)])
        return carry

    lax.fori_loop(0, RP // D_CH, dump, 0)


_BE = 8192


def _gate_body(a_ref, b_ref, g_ref, w_ref, m_ref):
    a = a_ref[...]
    b = b_ref[...]
    l = jnp.sum(a * b, axis=1, keepdims=True) + g_ref[...]
    y = 1.0 / (1.0 + jnp.exp(-l))
    w = jnp.where(y > THRESHOLD, y, 0.0) + 1e-7
    w_ref[...] = w
    m = a * w
    m_ref[0] = m[:, :32]
    m_ref[1] = m[:, 32:]


def _tc_gate(a, b, g2):
    grid = E_PAD // _BE
    return pl.pallas_call(
        _gate_body,
        grid=(grid,),
        in_specs=[
            pl.BlockSpec((_BE, D), lambda i: (i, 0)),
            pl.BlockSpec((_BE, D), lambda i: (i, 0)),
            pl.BlockSpec((_BE, 1), lambda i: (i, 0)),
        ],
        out_specs=[
            pl.BlockSpec((_BE, 1), lambda i: (i, 0)),
            pl.BlockSpec((2, _BE, 32), lambda i: (0, i, 0)),
        ],
        out_shape=[
            jax.ShapeDtypeStruct((E_PAD, 1), jnp.float32),
            jax.ShapeDtypeStruct((2, E_PAD, 32), jnp.float32),
        ],
    )(a, b, g2)


def _scale_body(a_ref, w_ref, m_ref):
    a = a_ref[...]
    m = a * w_ref[...]
    m_ref[0] = m[:, :32]
    m_ref[1] = m[:, 32:]


def _tc_scale(a, w2):
    grid = E_PAD // _BE
    return pl.pallas_call(
        _scale_body,
        grid=(grid,),
        in_specs=[
            pl.BlockSpec((_BE, D), lambda i: (i, 0)),
            pl.BlockSpec((_BE, 1), lambda i: (i, 0)),
        ],
        out_specs=pl.BlockSpec((2, _BE, 32), lambda i: (0, i, 0)),
        out_shape=jax.ShapeDtypeStruct((2, E_PAD, 32), jnp.float32),
    )(a, w2)


_BN = 6272


def _ep_body_notgt(ag_ref, w_ref, acc_ref, h_ref, out_ref):
    W = w_ref[...]
    o = ag_ref[0] @ W[:32, :] + ag_ref[1] @ W[32:, :]
    nrm = jnp.sqrt(jnp.sum(o * o, axis=1, keepdims=True))
    h = o / (nrm + 1e-12)
    h_ref[...] = h
    out_ref[...] = acc_ref[...] + h


def _ep_body_tgt(ag_ref, w_ref, acc_ref, tgt_ref, h_ref, out_ref):
    W = w_ref[...]
    o = ag_ref[0] @ W[:32, :] + ag_ref[1] @ W[32:, :] + tgt_ref[...] @ W
    nrm = jnp.sqrt(jnp.sum(o * o, axis=1, keepdims=True))
    h = o / (nrm + 1e-12)
    h_ref[...] = h
    out_ref[...] = acc_ref[...] + h


def _tc_epilogue(agg2, W, acc, tgt=None):
    grid = NP // _BN
    in_specs = [
        pl.BlockSpec((2, _BN, 32), lambda i: (0, i, 0)),
        pl.BlockSpec((D, D), lambda i: (0, 0)),
        pl.BlockSpec((_BN, D), lambda i: (i, 0)),
    ]
    args = [agg2, W, acc]
    body = _ep_body_notgt
    if tgt is not None:
        in_specs.append(pl.BlockSpec((_BN, D), lambda i: (i, 0)))
        args.append(tgt)
        body = _ep_body_tgt
    return pl.pallas_call(
        body,
        grid=(grid,),
        in_specs=in_specs,
        out_specs=[
            pl.BlockSpec((_BN, D), lambda i: (i, 0)),
            pl.BlockSpec((_BN, D), lambda i: (i, 0)),
        ],
        out_shape=[
            jax.ShapeDtypeStruct((NP, D), jnp.float32),
            jax.ShapeDtypeStruct((NP, D), jnp.float32),
        ],
    )(*args)


def _final_body(a_ref, b_ref, c_ref, d_ref, e_ref, f_ref, out_ref):
    out_ref[...] = (a_ref[...] + b_ref[...] + c_ref[...] + d_ref[...]
                    + e_ref[...] + f_ref[...]) * (1.0 / 6.0)


def _tc_final(xs):
    grid = NP // _BN
    spec = pl.BlockSpec((_BN, D), lambda i: (i, 0))
    return pl.pallas_call(
        _final_body,
        grid=(grid,),
        in_specs=[spec] * 6,
        out_specs=spec,
        out_shape=jax.ShapeDtypeStruct((NP, D), jnp.float32),
    )(*xs)


def _pad_edges(edge):
    src = jnp.pad(edge[0], (0, E_PAD - E))
    dst = jnp.pad(edge[1], (0, E_PAD - E), constant_values=NN)
    return src, dst, dst.reshape(E_PAD // IB, IB)


def _stage(x, src, dst, dst2, g2, W, acc_in, zeros, tgt=None, w_in=None):
    a = _sc_gather(x, src)
    w2 = None
    if w_in is None:
        b = _sc_gather(x, dst)
        w2, msg = _tc_gate(a, b, g2)
    else:
        msg = _tc_scale(a, w_in)
    agg2 = _sc_scatter(msg, dst2, zeros)
    h, acc = _tc_epilogue(agg2, W, acc_in, tgt)
    return h, acc, w2


def kernel(edge_ubg, edge_view, edge_cart, edge_buy, edge_view_buy,
           edge_cart_buy, user_emb, item_emb, W_ubg, W_view, W_cart, W_buy,
           W_view_buy, W_cb0, W_cb1, W_cb2):
    x0 = jnp.concatenate([user_emb, item_emb], axis=0)
    x0 = jnp.pad(x0, ((0, NP - NN), (0, 0)))
    zeros = jnp.zeros((NP, 32), jnp.float32)

    rk = jax.random.split(jax.random.key(42), 6)
    gs = []
    for i in range(6):
        e = jax.random.exponential(rk[i], (E,), dtype=jnp.float32)
        g = -jnp.log(e)
        gs.append(jnp.pad(g, (0, E_PAD - E))[:, None])

    edges = [_pad_edges(e) for e in (edge_ubg, edge_view, edge_cart, edge_buy,
                                     edge_view_buy, edge_cart_buy)]

    _, ubg, _ = _stage(x0, *edges[0], gs[0], W_ubg, x0, zeros)
    _, view, _ = _stage(ubg, *edges[1], gs[1], W_view, ubg, zeros)
    _, cart, _ = _stage(ubg, *edges[2], gs[2], W_cart, ubg, zeros)
    _, buy, _ = _stage(ubg, *edges[3], gs[3], W_buy, ubg, zeros)
    _, vb, _ = _stage(view, *edges[4], gs[4], W_view_buy, view, zeros)

    h, acc, w5 = _stage(cart, *edges[5], gs[5], W_cb0, cart, zeros, tgt=buy)
    h, acc, _ = _stage(h, *edges[5], None, W_cb1, acc, zeros, tgt=buy, w_in=w5)
    _, cb, _ = _stage(h, *edges[5], None, W_cb2, acc, zeros, tgt=buy, w_in=w5)

    final = _tc_final([ubg, view, cart, buy, vb, cb])
    return final[:NN]


# speedup vs baseline: 2.1085x; 2.1085x over previous
import functools

import jax
import jax.numpy as jnp
from jax import lax
from jax.experimental import pallas as pl
from jax.experimental.pallas import tpu as pltpu
from jax.experimental.pallas import tpu_sc as plsc

N_USERS = 25000
N_ITEMS = 25000
NN = N_USERS + 1 + N_ITEMS + 1
NP = 50176
D = 64
E = 800000
E_PAD = 819200
THRESHOLD = 0.05

NC, NS = 2, 16
NW = NC * NS
IB = 128

G_PER = E_PAD // NW
G_CH = 512
G_K = G_CH // IB
G_T = G_PER // G_CH
S_PER = E_PAD // NS
S_CH = 256
S_K = S_CH // IB
S_T = S_PER // S_CH
RP = NP // NS
D_CH = 224

_mesh = plsc.VectorSubcoreMesh(core_axis_name="c", subcore_axis_name="s")
_sc_params = pltpu.CompilerParams(use_tc_tiling_on_sc=False)
_sc_params_nl = pltpu.CompilerParams(use_tc_tiling_on_sc=False,
                                     needs_layout_passes=False)


LC = 256
NCH = E_PAD // LC
T_T = G_PER // LC


def _allsum(v, lane):
    tot0 = jnp.where(lane == 0, lax.rev(plsc.cumsum(v), (0,)), 0.0)
    return plsc.cumsum(tot0)


def _learner_compute(sdg_v, a_v, b_v, m_v, wr_v, bslot, t):
    def group(k, carry):
        base = k * 16
        gv16 = plsc.bitcast(sdg_v[bslot, 2, pl.ds(base, 16)], jnp.float32)
        lane = lax.iota(jnp.int32, 16)
        for e in range(16):
            off = base + e
            av = [a_v[bslot, off, pl.ds(q * 16, 16)] for q in range(4)]
            acc = av[0] * b_v[bslot, off, pl.ds(0, 16)]
            for q in range(1, 4):
                acc = acc + av[q] * b_v[bslot, off, pl.ds(q * 16, 16)]
            lv = _allsum(acc + jnp.where(lane == e, gv16, 0.0), lane)
            y = 1.0 / (1.0 + jnp.exp(-lv))
            w16 = jnp.where(y > THRESHOLD, y, 0.0) + 1e-7
            wr_v[bslot, pl.ds(off * 16, 16)] = w16
            for q in range(4):
                mval = av[q] * w16
                if q < 2:
                    m_v[bslot, 0, off, pl.ds(q * 16, 16)] = mval
                else:
                    m_v[bslot, 1, off, pl.ds((q - 2) * 16, 16)] = mval
        return carry

    lax.fori_loop(0, LC // 16, group, 0)


@functools.partial(
    pl.kernel, mesh=_mesh, compiler_params=_sc_params_nl,
    out_type=[
        jax.ShapeDtypeStruct((NCH, 2, LC, 32), jnp.float32),
        jax.ShapeDtypeStruct((E_PAD * 16,), jnp.float32),
    ],
    scratch_types=[
        pltpu.VMEM((2, 3, LC), jnp.int32),
        pltpu.VMEM((2, LC, D), jnp.float32),
        pltpu.VMEM((2, LC, D), jnp.float32),
        pltpu.VMEM((2, 2, LC, 32), jnp.float32),
        pltpu.VMEM((2, LC * 16), jnp.float32),
        pltpu.SemaphoreType.DMA((2,)),
        pltpu.SemaphoreType.DMA((2,)),
    ],
)
def _sc_learner(table_hbm, sdg_hbm, msg_hbm, w_hbm, sdg_v, a_v, b_v, m_v,
                wr_v, gsem, wsem):
    wid = lax.axis_index("s") * NC + lax.axis_index("c")
    cbase = wid * T_T
    ebase = wid * G_PER

    def fire(t, b):
        pltpu.sync_copy(sdg_hbm.at[cbase + t], sdg_v.at[b])
        pltpu.async_copy(table_hbm.at[sdg_v.at[b].at[0]], a_v.at[b],
                         gsem.at[b])
        pltpu.async_copy(table_hbm.at[sdg_v.at[b].at[1]], b_v.at[b],
                         gsem.at[b])

    def drain_gathers(b):
        pltpu.make_async_copy(table_hbm.at[pl.ds(0, LC)], a_v.at[b],
                              gsem.at[b]).wait()
        pltpu.make_async_copy(table_hbm.at[pl.ds(0, LC)], b_v.at[b],
                              gsem.at[b]).wait()

    def drain_wb(b):
        pltpu.make_async_copy(m_v.at[b], msg_hbm.at[cbase], wsem.at[b]).wait()
        pltpu.make_async_copy(wr_v.at[b], w_hbm.at[pl.ds(0, LC * 16)],
                              wsem.at[b]).wait()

    def process(t, b, first):
        drain_gathers(b)
        if not first:
            drain_wb(b)
        _learner_compute(sdg_v, a_v, b_v, m_v, wr_v, b, t)
        pltpu.async_copy(m_v.at[b], msg_hbm.at[cbase + t], wsem.at[b])
        pltpu.async_copy(wr_v.at[b],
                         w_hbm.at[pl.ds((ebase + t * LC) * 16, LC * 16)],
                         wsem.at[b])

    fire(0, 0)
    fire(1, 1)
    process(0, 0, True)
    fire(2, 0)
    process(1, 1, True)
    fire(3, 1)

    def body(tt, carry):
        t = 2 * tt
        process(t, 0, False)
        fire(lax.rem(t + 2, T_T), 0)
        process(t + 1, 1, False)
        fire(lax.rem(t + 3, T_T), 1)
        return carry

    lax.fori_loop(1, T_T // 2, body, 0)
    drain_gathers(0)
    drain_gathers(1)
    drain_wb(0)
    drain_wb(1)


def _scale_compute(a_v, m_v, wr_v, bslot):
    def group(k, carry):
        base = k * 16
        for e in range(16):
            off = base + e
            w16 = wr_v[bslot, pl.ds(off * 16, 16)]
            for q in range(4):
                mval = a_v[bslot, off, pl.ds(q * 16, 16)] * w16
                if q < 2:
                    m_v[bslot, 0, off, pl.ds(q * 16, 16)] = mval
                else:
                    m_v[bslot, 1, off, pl.ds((q - 2) * 16, 16)] = mval
        return carry

    lax.fori_loop(0, LC // 16, group, 0)


@functools.partial(
    pl.kernel, mesh=_mesh, compiler_params=_sc_params_nl,
    out_type=jax.ShapeDtypeStruct((NCH, 2, LC, 32), jnp.float32),
    scratch_types=[
        pltpu.VMEM((2, 3, LC), jnp.int32),
        pltpu.VMEM((2, LC, D), jnp.float32),
        pltpu.VMEM((2, LC * 16), jnp.float32),
        pltpu.VMEM((2, 2, LC, 32), jnp.float32),
        pltpu.SemaphoreType.DMA((2,)),
        pltpu.SemaphoreType.DMA((2,)),
    ],
)
def _sc_scale(table_hbm, sdg_hbm, w_hbm, msg_hbm, sdg_v, a_v, wr_v, m_v,
              gsem, wsem):
    wid = lax.axis_index("s") * NC + lax.axis_index("c")
    cbase = wid * T_T
    ebase = wid * G_PER

    def fire(t, b):
        pltpu.sync_copy(sdg_hbm.at[cbase + t], sdg_v.at[b])
        pltpu.sync_copy(w_hbm.at[pl.ds((ebase + t * LC) * 16, LC * 16)],
                        wr_v.at[b])
        pltpu.async_copy(table_hbm.at[sdg_v.at[b].at[0]], a_v.at[b],
                         gsem.at[b])

    def drain_gathers(b):
        pltpu.make_async_copy(table_hbm.at[pl.ds(0, LC)], a_v.at[b],
                              gsem.at[b]).wait()

    def drain_wb(b):
        pltpu.make_async_copy(m_v.at[b], msg_hbm.at[cbase], wsem.at[b]).wait()

    def process(t, b, first):
        drain_gathers(b)
        if not first:
            drain_wb(b)
        _scale_compute(a_v, m_v, wr_v, b)
        pltpu.async_copy(m_v.at[b], msg_hbm.at[cbase + t], wsem.at[b])

    fire(0, 0)
    fire(1, 1)
    process(0, 0, True)
    fire(2, 0)
    process(1, 1, True)
    fire(3, 1)

    def body(tt, carry):
        t = 2 * tt
        process(t, 0, False)
        fire(lax.rem(t + 2, T_T), 0)
        process(t + 1, 1, False)
        fire(lax.rem(t + 3, T_T), 1)
        return carry

    lax.fori_loop(1, T_T // 2, body, 0)
    drain_gathers(0)
    drain_gathers(1)
    drain_wb(0)
    drain_wb(1)


@functools.partial(
    pl.kernel, mesh=_mesh, compiler_params=_sc_params,
    out_type=jax.ShapeDtypeStruct((NC, NP, 32), jnp.float32),
    scratch_types=[
        pltpu.VMEM((2, S_K, IB), jnp.int32),
        pltpu.VMEM((2, S_CH, 32), jnp.float32),
        pltpu.VMEM_SHARED((NP, 32), jnp.float32),
        pltpu.SemaphoreType.DMA((2,)),
        pltpu.SemaphoreType.DMA((2,)),
    ],
)
def _sc_scatter(msg_hbm, dst2_hbm, zero_hbm, out_hbm, idx_v, msg_v,
                shared, lsem, ssem):
    c = lax.axis_index("c")
    s = lax.axis_index("s")
    chbase = s * (S_PER // LC)
    pltpu.sync_copy(zero_hbm.at[pl.ds(s * RP, RP)], shared.at[pl.ds(s * RP, RP)])
    plsc.subcore_barrier()

    rbase = s * (S_PER // IB)
    ebase = s * S_PER

    def fire(g, b):
        pltpu.sync_copy(dst2_hbm.at[pl.ds(rbase + g * S_K, S_K)], idx_v.at[b])
        pltpu.async_copy(msg_hbm.at[chbase + g, c], msg_v.at[b], lsem.at[b])

    def process(b):
        pltpu.make_async_copy(msg_hbm.at[chbase, c], msg_v.at[b],
                              lsem.at[b]).wait()
        for j in range(S_K):
            pltpu.async_copy(msg_v.at[b].at[pl.ds(j * IB, IB)],
                             shared.at[idx_v.at[b].at[j]], ssem.at[b],
                             add=True)
        for j in range(S_K):
            pltpu.make_async_copy(msg_v.at[b].at[pl.ds(j * IB, IB)],
                                  shared.at[idx_v.at[b].at[j]],
                                  ssem.at[b]).wait()

    fire(0, 0)

    def body(t, carry):
        g = 2 * t
        fire(g + 1, 1)
        process(0)
        fire(lax.rem(g + 2, S_T), 0)
        process(1)
        return carry

    lax.fori_loop(0, S_T // 2, body, 0)
    pltpu.make_async_copy(msg_hbm.at[chbase, c], msg_v.at[0],
                          lsem.at[0]).wait()
    plsc.subcore_barrier()

    def dump(jj, carry):
        r = s * RP + jj * D_CH
        pltpu.sync_copy(shared.at[pl.ds(r, D_CH)], msg_v.at[0, pl.ds(0, D_CH)])
        pltpu.sync_copy(msg_v.at[0, pl.ds(0, D_CH)],
                        out_hbm.at[c, pl.ds(r, D_CH)])
        return carry

    lax.fori_loop(0, RP // D_CH, dump, 0)


_BN = 6272


def _ep_body_notgt(ag_ref, w_ref, acc_ref, h_ref, out_ref):
    W = w_ref[...]
    o = ag_ref[0] @ W[:32, :] + ag_ref[1] @ W[32:, :]
    nrm = jnp.sqrt(jnp.sum(o * o, axis=1, keepdims=True))
    h = o / (nrm + 1e-12)
    h_ref[...] = h
    out_ref[...] = acc_ref[...] + h


def _ep_body_tgt(ag_ref, w_ref, acc_ref, tgt_ref, h_ref, out_ref):
    W = w_ref[...]
    o = ag_ref[0] @ W[:32, :] + ag_ref[1] @ W[32:, :] + tgt_ref[...] @ W
    nrm = jnp.sqrt(jnp.sum(o * o, axis=1, keepdims=True))
    h = o / (nrm + 1e-12)
    h_ref[...] = h
    out_ref[...] = acc_ref[...] + h


def _tc_epilogue(agg2, W, acc, tgt=None):
    grid = NP // _BN
    in_specs = [
        pl.BlockSpec((2, _BN, 32), lambda i: (0, i, 0)),
        pl.BlockSpec((D, D), lambda i: (0, 0)),
        pl.BlockSpec((_BN, D), lambda i: (i, 0)),
    ]
    args = [agg2, W, acc]
    body = _ep_body_notgt
    if tgt is not None:
        in_specs.append(pl.BlockSpec((_BN, D), lambda i: (i, 0)))
        args.append(tgt)
        body = _ep_body_tgt
    return pl.pallas_call(
        body,
        grid=(grid,),
        in_specs=in_specs,
        out_specs=[
            pl.BlockSpec((_BN, D), lambda i: (i, 0)),
            pl.BlockSpec((_BN, D), lambda i: (i, 0)),
        ],
        out_shape=[
            jax.ShapeDtypeStruct((NP, D), jnp.float32),
            jax.ShapeDtypeStruct((NP, D), jnp.float32),
        ],
    )(*args)


def _final_body(a_ref, b_ref, c_ref, d_ref, e_ref, f_ref, out_ref):
    out_ref[...] = (a_ref[...] + b_ref[...] + c_ref[...] + d_ref[...]
                    + e_ref[...] + f_ref[...]) * (1.0 / 6.0)


def _tc_final(xs):
    grid = NP // _BN
    spec = pl.BlockSpec((_BN, D), lambda i: (i, 0))
    return pl.pallas_call(
        _final_body,
        grid=(grid,),
        in_specs=[spec] * 6,
        out_specs=spec,
        out_shape=jax.ShapeDtypeStruct((NP, D), jnp.float32),
    )(*xs)


def _gumbel_noise():
    rk = jax.random.split(jax.random.key(42), 6)
    gs = []
    for i in range(6):
        e = jax.random.exponential(rk[i], (E,), dtype=jnp.float32)
        g = -jnp.log(e)
        gs.append(jnp.pad(g, (0, E_PAD - E)))
    return gs


def _pack_edges(edge, g1d):
    src = jnp.pad(edge[0], (0, E_PAD - E))
    dst = jnp.pad(edge[1], (0, E_PAD - E), constant_values=NN)
    gi = jax.lax.bitcast_convert_type(g1d, jnp.int32)
    sdg = jnp.stack([src.reshape(NCH, LC), dst.reshape(NCH, LC),
                     gi.reshape(NCH, LC)], axis=1)
    return sdg, dst.reshape(E_PAD // IB, IB)


def _stage(x, sdg, dst2, W, acc_in, zeros, tgt=None, w_in=None):
    w2 = None
    if w_in is None:
        msg, w2 = _sc_learner(x, sdg)
    else:
        msg = _sc_scale(x, sdg, w_in)
    agg2 = _sc_scatter(msg, dst2, zeros)
    h, acc = _tc_epilogue(agg2, W, acc_in, tgt)
    return h, acc, w2


def kernel(edge_ubg, edge_view, edge_cart, edge_buy, edge_view_buy,
           edge_cart_buy, user_emb, item_emb, W_ubg, W_view, W_cart, W_buy,
           W_view_buy, W_cb0, W_cb1, W_cb2):
    x0 = jnp.concatenate([user_emb, item_emb], axis=0)
    x0 = jnp.pad(x0, ((0, NP - NN), (0, 0)))
    zeros = jnp.zeros((NP, 32), jnp.float32)

    ebs = (edge_ubg, edge_view, edge_cart, edge_buy, edge_view_buy,
           edge_cart_buy)
    gs = _gumbel_noise()
    edges = [_pack_edges(e, gs[i]) for i, e in enumerate(ebs)]

    _, ubg, _ = _stage(x0, *edges[0], W_ubg, x0, zeros)
    _, view, _ = _stage(ubg, *edges[1], W_view, ubg, zeros)
    _, cart, _ = _stage(ubg, *edges[2], W_cart, ubg, zeros)
    _, buy, _ = _stage(ubg, *edges[3], W_buy, ubg, zeros)
    _, vb, _ = _stage(view, *edges[4], W_view_buy, view, zeros)

    h, acc, w5 = _stage(cart, *edges[5], W_cb0, cart, zeros, tgt=buy)
    h, acc, _ = _stage(h, *edges[5], W_cb1, acc, zeros, tgt=buy, w_in=w5)
    _, cb, _ = _stage(h, *edges[5], W_cb2, acc, zeros, tgt=buy, w_in=w5)

    final = _tc_final([ubg, view, cart, buy, vb, cb])
    return final[:NN]


# speedup vs baseline: 2.1610x; 1.0249x over previous
import functools

import jax
import jax.numpy as jnp
from jax import lax
from jax.experimental import pallas as pl
from jax.experimental.pallas import tpu as pltpu
from jax.experimental.pallas import tpu_sc as plsc

N_USERS = 25000
N_ITEMS = 25000
NN = N_USERS + 1 + N_ITEMS + 1
NP = 50176
D = 64
E = 800000
E_PAD = 819200
THRESHOLD = 0.05

NC, NS = 2, 16
NW = NC * NS
IB = 128

G_PER = E_PAD // NW
G_CH = 512
G_K = G_CH // IB
G_T = G_PER // G_CH
S_PER = E_PAD // NS
S_CH = 256
S_K = S_CH // IB
S_T = S_PER // S_CH
RP = NP // NS
D_CH = 224

_mesh = plsc.VectorSubcoreMesh(core_axis_name="c", subcore_axis_name="s")
_sc_params = pltpu.CompilerParams(use_tc_tiling_on_sc=False)
_sc_params_nl = pltpu.CompilerParams(use_tc_tiling_on_sc=False,
                                     needs_layout_passes=False)


LC = 256
NCH = E_PAD // LC
T_T = G_PER // LC


def _allsum(v, lane):
    tot0 = jnp.where(lane == 0, lax.rev(plsc.cumsum(v), (0,)), 0.0)
    return plsc.cumsum(tot0)


def _learner_compute(sdg_v, a_v, b_v, m_v, wr_v, bslot, t):
    def group(k, carry):
        base = k * 16
        gv16 = plsc.bitcast(sdg_v[bslot, 2, pl.ds(base, 16)], jnp.float32)
        lane = lax.iota(jnp.int32, 16)
        for e in range(16):
            off = base + e
            av = [a_v[bslot, off, pl.ds(q * 16, 16)] for q in range(4)]
            acc = av[0] * b_v[bslot, off, pl.ds(0, 16)]
            for q in range(1, 4):
                acc = acc + av[q] * b_v[bslot, off, pl.ds(q * 16, 16)]
            lv = _allsum(acc + jnp.where(lane == e, gv16, 0.0), lane)
            y = 1.0 / (1.0 + jnp.exp(-lv))
            w16 = jnp.where(y > THRESHOLD, y, 0.0) + 1e-7
            wr_v[bslot, pl.ds(off * 16, 16)] = w16
            for q in range(4):
                mval = av[q] * w16
                if q < 2:
                    m_v[bslot, 0, off, pl.ds(q * 16, 16)] = mval
                else:
                    m_v[bslot, 1, off, pl.ds((q - 2) * 16, 16)] = mval
        return carry

    lax.fori_loop(0, LC // 16, group, 0)


def _make_learner(with_w):
    out_type = jax.ShapeDtypeStruct((NCH, 2, LC, 32), jnp.float32)
    if with_w:
        out_type = [out_type,
                    jax.ShapeDtypeStruct((E_PAD * 16,), jnp.float32)]

    @functools.partial(
        pl.kernel, mesh=_mesh, compiler_params=_sc_params_nl,
        out_type=out_type,
        scratch_types=[
            pltpu.VMEM((2, 3, LC), jnp.int32),
            pltpu.VMEM((2, LC, D), jnp.float32),
            pltpu.VMEM((2, LC, D), jnp.float32),
            pltpu.VMEM((2, 2, LC, 32), jnp.float32),
            pltpu.VMEM((2, LC * 16), jnp.float32),
            pltpu.SemaphoreType.DMA((2,)),
            pltpu.SemaphoreType.DMA((2,)),
        ],
    )
    def _lk(table_hbm, sdg_hbm, msg_hbm, *rest):
        if with_w:
            (w_hbm, sdg_v, a_v, b_v, m_v, wr_v, gsem, wsem) = rest
        else:
            (sdg_v, a_v, b_v, m_v, wr_v, gsem, wsem) = rest
        wid = lax.axis_index("s") * NC + lax.axis_index("c")
        cbase = wid * T_T
        ebase = wid * G_PER

        def fire(t, b):
            pltpu.sync_copy(sdg_hbm.at[cbase + t], sdg_v.at[b])
            pltpu.async_copy(table_hbm.at[sdg_v.at[b].at[0]], a_v.at[b],
                             gsem.at[b])
            pltpu.async_copy(table_hbm.at[sdg_v.at[b].at[1]], b_v.at[b],
                             gsem.at[b])

        def drain_gathers(b):
            pltpu.make_async_copy(table_hbm.at[pl.ds(0, LC)], a_v.at[b],
                                  gsem.at[b]).wait()
            pltpu.make_async_copy(table_hbm.at[pl.ds(0, LC)], b_v.at[b],
                                  gsem.at[b]).wait()

        def drain_wb(b):
            pltpu.make_async_copy(m_v.at[b], msg_hbm.at[cbase],
                                  wsem.at[b]).wait()
            if with_w:
                pltpu.make_async_copy(wr_v.at[b], w_hbm.at[pl.ds(0, LC * 16)],
                                      wsem.at[b]).wait()

        def process(t, b, first):
            drain_gathers(b)
            if not first:
                drain_wb(b)
            _learner_compute(sdg_v, a_v, b_v, m_v, wr_v, b, t)
            pltpu.async_copy(m_v.at[b], msg_hbm.at[cbase + t], wsem.at[b])
            if with_w:
                pltpu.async_copy(
                    wr_v.at[b],
                    w_hbm.at[pl.ds((ebase + t * LC) * 16, LC * 16)],
                    wsem.at[b])

        fire(0, 0)
        fire(1, 1)
        process(0, 0, True)
        fire(2, 0)
        process(1, 1, True)
        fire(3, 1)

        def body(tt, carry):
            t = 2 * tt
            process(t, 0, False)
            fire(lax.rem(t + 2, T_T), 0)
            process(t + 1, 1, False)
            fire(lax.rem(t + 3, T_T), 1)
            return carry

        lax.fori_loop(1, T_T // 2, body, 0)
        drain_gathers(0)
        drain_gathers(1)
        drain_wb(0)
        drain_wb(1)

    return _lk


_sc_learner_w = _make_learner(True)
_sc_learner_nw = _make_learner(False)


def _scale_compute(a_v, m_v, wr_v, bslot):
    def group(k, carry):
        base = k * 16
        for e in range(16):
            off = base + e
            w16 = wr_v[bslot, pl.ds(off * 16, 16)]
            for q in range(4):
                mval = a_v[bslot, off, pl.ds(q * 16, 16)] * w16
                if q < 2:
                    m_v[bslot, 0, off, pl.ds(q * 16, 16)] = mval
                else:
                    m_v[bslot, 1, off, pl.ds((q - 2) * 16, 16)] = mval
        return carry

    lax.fori_loop(0, LC // 16, group, 0)


@functools.partial(
    pl.kernel, mesh=_mesh, compiler_params=_sc_params_nl,
    out_type=jax.ShapeDtypeStruct((NCH, 2, LC, 32), jnp.float32),
    scratch_types=[
        pltpu.VMEM((2, 3, LC), jnp.int32),
        pltpu.VMEM((2, LC, D), jnp.float32),
        pltpu.VMEM((2, LC * 16), jnp.float32),
        pltpu.VMEM((2, 2, LC, 32), jnp.float32),
        pltpu.SemaphoreType.DMA((2,)),
        pltpu.SemaphoreType.DMA((2,)),
    ],
)
def _sc_scale(table_hbm, sdg_hbm, w_hbm, msg_hbm, sdg_v, a_v, wr_v, m_v,
              gsem, wsem):
    wid = lax.axis_index("s") * NC + lax.axis_index("c")
    cbase = wid * T_T
    ebase = wid * G_PER

    def fire(t, b):
        pltpu.sync_copy(sdg_hbm.at[cbase + t], sdg_v.at[b])
        pltpu.sync_copy(w_hbm.at[pl.ds((ebase + t * LC) * 16, LC * 16)],
                        wr_v.at[b])
        pltpu.async_copy(table_hbm.at[sdg_v.at[b].at[0]], a_v.at[b],
                         gsem.at[b])

    def drain_gathers(b):
        pltpu.make_async_copy(table_hbm.at[pl.ds(0, LC)], a_v.at[b],
                              gsem.at[b]).wait()

    def drain_wb(b):
        pltpu.make_async_copy(m_v.at[b], msg_hbm.at[cbase], wsem.at[b]).wait()

    def process(t, b, first):
        drain_gathers(b)
        if not first:
            drain_wb(b)
        _scale_compute(a_v, m_v, wr_v, b)
        pltpu.async_copy(m_v.at[b], msg_hbm.at[cbase + t], wsem.at[b])

    fire(0, 0)
    fire(1, 1)
    process(0, 0, True)
    fire(2, 0)
    process(1, 1, True)
    fire(3, 1)

    def body(tt, carry):
        t = 2 * tt
        process(t, 0, False)
        fire(lax.rem(t + 2, T_T), 0)
        process(t + 1, 1, False)
        fire(lax.rem(t + 3, T_T), 1)
        return carry

    lax.fori_loop(1, T_T // 2, body, 0)
    drain_gathers(0)
    drain_gathers(1)
    drain_wb(0)
    drain_wb(1)


@functools.partial(
    pl.kernel, mesh=_mesh, compiler_params=_sc_params,
    out_type=jax.ShapeDtypeStruct((NC, NP, 32), jnp.float32),
    scratch_types=[
        pltpu.VMEM((2, S_K, IB), jnp.int32),
        pltpu.VMEM((2, S_CH, 32), jnp.float32),
        pltpu.VMEM_SHARED((NP, 32), jnp.float32),
        pltpu.SemaphoreType.DMA((2,)),
        pltpu.SemaphoreType.DMA((2,)),
    ],
)
def _sc_scatter(msg_hbm, dst2_hbm, zero_hbm, out_hbm, idx_v, msg_v,
                shared, lsem, ssem):
    c = lax.axis_index("c")
    s = lax.axis_index("s")
    chbase = s * (S_PER // LC)
    pltpu.sync_copy(zero_hbm.at[pl.ds(s * RP, RP)], shared.at[pl.ds(s * RP, RP)])
    plsc.subcore_barrier()

    rbase = s * (S_PER // IB)
    ebase = s * S_PER

    def fire(g, b):
        pltpu.sync_copy(dst2_hbm.at[pl.ds(rbase + g * S_K, S_K)], idx_v.at[b])
        pltpu.async_copy(msg_hbm.at[chbase + g, c], msg_v.at[b], lsem.at[b])

    def process(b):
        pltpu.make_async_copy(msg_hbm.at[chbase, c], msg_v.at[b],
                              lsem.at[b]).wait()
        for j in range(S_K):
            pltpu.async_copy(msg_v.at[b].at[pl.ds(j * IB, IB)],
                             shared.at[idx_v.at[b].at[j]], ssem.at[b],
                             add=True)
        for j in range(S_K):
            pltpu.make_async_copy(msg_v.at[b].at[pl.ds(j * IB, IB)],
                                  shared.at[idx_v.at[b].at[j]],
                                  ssem.at[b]).wait()

    fire(0, 0)

    def body(t, carry):
        g = 2 * t
        fire(g + 1, 1)
        process(0)
        fire(lax.rem(g + 2, S_T), 0)
        process(1)
        return carry

    lax.fori_loop(0, S_T // 2, body, 0)
    pltpu.make_async_copy(msg_hbm.at[chbase, c], msg_v.at[0],
                          lsem.at[0]).wait()
    plsc.subcore_barrier()

    def dump(jj, carry):
        r = s * RP + jj * D_CH
        pltpu.sync_copy(shared.at[pl.ds(r, D_CH)], msg_v.at[0, pl.ds(0, D_CH)])
        pltpu.sync_copy(msg_v.at[0, pl.ds(0, D_CH)],
                        out_hbm.at[c, pl.ds(r, D_CH)])
        return carry

    lax.fori_loop(0, RP // D_CH, dump, 0)


_BN = 6272


def _ep_body_notgt(ag_ref, w_ref, acc_ref, h_ref, out_ref):
    W = w_ref[...]
    o = ag_ref[0] @ W[:32, :] + ag_ref[1] @ W[32:, :]
    nrm = jnp.sqrt(jnp.sum(o * o, axis=1, keepdims=True))
    h = o / (nrm + 1e-12)
    h_ref[...] = h
    out_ref[...] = acc_ref[...] + h


def _ep_body_tgt(ag_ref, w_ref, acc_ref, tgt_ref, h_ref, out_ref):
    W = w_ref[...]
    o = ag_ref[0] @ W[:32, :] + ag_ref[1] @ W[32:, :] + tgt_ref[...] @ W
    nrm = jnp.sqrt(jnp.sum(o * o, axis=1, keepdims=True))
    h = o / (nrm + 1e-12)
    h_ref[...] = h
    out_ref[...] = acc_ref[...] + h


def _tc_epilogue(agg2, W, acc, tgt=None):
    grid = NP // _BN
    in_specs = [
        pl.BlockSpec((2, _BN, 32), lambda i: (0, i, 0)),
        pl.BlockSpec((D, D), lambda i: (0, 0)),
        pl.BlockSpec((_BN, D), lambda i: (i, 0)),
    ]
    args = [agg2, W, acc]
    body = _ep_body_notgt
    if tgt is not None:
        in_specs.append(pl.BlockSpec((_BN, D), lambda i: (i, 0)))
        args.append(tgt)
        body = _ep_body_tgt
    return pl.pallas_call(
        body,
        grid=(grid,),
        in_specs=in_specs,
        out_specs=[
            pl.BlockSpec((_BN, D), lambda i: (i, 0)),
            pl.BlockSpec((_BN, D), lambda i: (i, 0)),
        ],
        out_shape=[
            jax.ShapeDtypeStruct((NP, D), jnp.float32),
            jax.ShapeDtypeStruct((NP, D), jnp.float32),
        ],
    )(*args)


def _final_body(a_ref, b_ref, c_ref, d_ref, e_ref, f_ref, out_ref):
    out_ref[...] = (a_ref[...] + b_ref[...] + c_ref[...] + d_ref[...]
                    + e_ref[...] + f_ref[...]) * (1.0 / 6.0)


def _tc_final(xs):
    grid = NP // _BN
    spec = pl.BlockSpec((_BN, D), lambda i: (i, 0))
    return pl.pallas_call(
        _final_body,
        grid=(grid,),
        in_specs=[spec] * 6,
        out_specs=spec,
        out_shape=jax.ShapeDtypeStruct((NP, D), jnp.float32),
    )(*xs)


def _gumbel_noise():
    rk = jax.random.split(jax.random.key(42), 6)
    gs = []
    for i in range(6):
        e = jax.random.exponential(rk[i], (E,), dtype=jnp.float32)
        g = -jnp.log(e)
        gs.append(jnp.pad(g, (0, E_PAD - E)))
    return gs


def _pack_edges(edge, g1d):
    src = jnp.pad(edge[0], (0, E_PAD - E))
    dst = jnp.pad(edge[1], (0, E_PAD - E), constant_values=NN)
    gi = jax.lax.bitcast_convert_type(g1d, jnp.int32)
    sdg = jnp.stack([src.reshape(NCH, LC), dst.reshape(NCH, LC),
                     gi.reshape(NCH, LC)], axis=1)
    return sdg, dst.reshape(E_PAD // IB, IB)


def _stage(x, sdg, dst2, W, acc_in, zeros, tgt=None, w_in=None, need_w=False):
    w2 = None
    if w_in is None:
        if need_w:
            msg, w2 = _sc_learner_w(x, sdg)
        else:
            msg = _sc_learner_nw(x, sdg)
    else:
        msg = _sc_scale(x, sdg, w_in)
    agg2 = _sc_scatter(msg, dst2, zeros)
    h, acc = _tc_epilogue(agg2, W, acc_in, tgt)
    return h, acc, w2


def kernel(edge_ubg, edge_view, edge_cart, edge_buy, edge_view_buy,
           edge_cart_buy, user_emb, item_emb, W_ubg, W_view, W_cart, W_buy,
           W_view_buy, W_cb0, W_cb1, W_cb2):
    x0 = jnp.concatenate([user_emb, item_emb], axis=0)
    x0 = jnp.pad(x0, ((0, NP - NN), (0, 0)))
    zeros = jnp.zeros((NP, 32), jnp.float32)

    ebs = (edge_ubg, edge_view, edge_cart, edge_buy, edge_view_buy,
           edge_cart_buy)
    gs = _gumbel_noise()
    edges = [_pack_edges(e, gs[i]) for i, e in enumerate(ebs)]

    _, ubg, _ = _stage(x0, *edges[0], W_ubg, x0, zeros)
    _, view, _ = _stage(ubg, *edges[1], W_view, ubg, zeros)
    _, cart, _ = _stage(ubg, *edges[2], W_cart, ubg, zeros)
    _, buy, _ = _stage(ubg, *edges[3], W_buy, ubg, zeros)
    _, vb, _ = _stage(view, *edges[4], W_view_buy, view, zeros)

    h, acc, w5 = _stage(cart, *edges[5], W_cb0, cart, zeros, tgt=buy,
                        need_w=True)
    h, acc, _ = _stage(h, *edges[5], W_cb1, acc, zeros, tgt=buy, w_in=w5)
    _, cb, _ = _stage(h, *edges[5], W_cb2, acc, zeros, tgt=buy, w_in=w5)

    final = _tc_final([ubg, view, cart, buy, vb, cb])
    return final[:NN]
